# baseline (speedup 1.0000x reference)
import functools

import jax
import jax.numpy as jnp
from jax import lax
from jax.experimental import pallas as pl
from jax.experimental.pallas import tpu as pltpu
from jax.experimental.pallas import tpu_sc as plsc

V, D, N, C = 1000000, 64, 16384, 20

NC, NS = 2, 16
NW = NC * NS
WORDS_PER_W = N // NW
W = 32
CW = W * C
IB = 128
NB = CW // IB
NCHUNK = WORDS_PER_W // W
NREG = D // 16


def _sc_body(words_hbm, pc_hbm, nc_hbm, emb_hbm, out_hbm,
             ps_hbm, ns_hbm,
             widx_v, tbuf_v, pidx_v, pbuf_v, nidx_v, nbuf_v,
             psb_v, nsb_v, sem_t, sem_p, sem_n, sem_s):
    wid = lax.axis_index("s") * NC + lax.axis_index("c")
    base = wid * WORDS_PER_W

    def chunk_body(g, _):
        wbase = base + g * W
        cbase = wbase * C

        pltpu.sync_copy(words_hbm.at[pl.ds(wbase, W)], widx_v)
        for j in range(NB):
            pltpu.sync_copy(pc_hbm.at[pl.ds(cbase + j * IB, IB)], pidx_v.at[j])
            pltpu.sync_copy(nc_hbm.at[pl.ds(cbase + j * IB, IB)], nidx_v.at[j])

        cp_t = pltpu.async_copy(emb_hbm.at[widx_v], tbuf_v, sem_t)
        cps = []
        for j in range(NB):
            cps.append(pltpu.async_copy(
                out_hbm.at[pidx_v.at[j]], pbuf_v.at[pl.ds(j * IB, IB)], sem_p))
            cps.append(pltpu.async_copy(
                out_hbm.at[nidx_v.at[j]], nbuf_v.at[pl.ds(j * IB, IB)], sem_n))
        cp_t.wait()
        for cp in cps:
            cp.wait()

        lanes = lax.iota(jnp.int32, 16)
        perms = [jnp.bitwise_xor(lanes, k) for k in (1, 2, 4, 8)]

        def allsum(v):
            for p in perms:
                v = v + jnp.take_along_axis(v, p, axis=0)
            return v

        def word_body(w, carry):
            ps_vec, ns_vec = carry
            r0 = w * C
            psum = []
            nsum = []
            for j in range(NREG):
                sl = pl.ds(j * 16, 16)
                ps_acc = pbuf_v[r0, sl]
                ns_acc = nbuf_v[r0, sl]
                for c in range(1, C):
                    ps_acc = ps_acc + pbuf_v[r0 + c, sl]
                    ns_acc = ns_acc + nbuf_v[r0 + c, sl]
                t = tbuf_v[w, sl]
                psum.append(ps_acc * t)
                nsum.append(ns_acc * t)
            pq = (psum[0] + psum[1]) + (psum[2] + psum[3])
            nq = (nsum[0] + nsum[1]) + (nsum[2] + nsum[3])
            lane = lax.rem(w, 16)
            ps_vec = jnp.where(lanes == lane, allsum(pq), ps_vec)
            ns_vec = jnp.where(lanes == lane, allsum(nq), ns_vec)

            @pl.when(lane == 15)
            def _():
                off = (w // 16) * 16
                psb_v[pl.ds(off, 16)] = ps_vec
                nsb_v[pl.ds(off, 16)] = ns_vec

            return ps_vec, ns_vec

        zv = jnp.zeros((16,), jnp.float32)
        lax.fori_loop(0, W, word_body, (zv, zv), unroll=False)

        pltpu.sync_copy(psb_v, ps_hbm.at[pl.ds(wbase, W)])
        pltpu.sync_copy(nsb_v, ns_hbm.at[pl.ds(wbase, W)])
        return 0

    lax.fori_loop(0, NCHUNK, chunk_body, 0, unroll=False)


@jax.jit
def _sc_sims(words, pc2d, nc2d, emb_table, out_table):
    mesh = plsc.VectorSubcoreMesh(core_axis_name="c", subcore_axis_name="s",
                                  num_cores=NC, num_subcores=NS)
    f = pl.kernel(
        _sc_body,
        out_type=(jax.ShapeDtypeStruct((N,), jnp.float32),
                  jax.ShapeDtypeStruct((N,), jnp.float32)),
        mesh=mesh,
        compiler_params=pltpu.CompilerParams(use_tc_tiling_on_sc=False),
        scratch_types=[
            pltpu.VMEM((W,), jnp.int32),
            pltpu.VMEM((W, D), jnp.float32),
            pltpu.VMEM((NB, IB), jnp.int32),
            pltpu.VMEM((CW, D), jnp.float32),
            pltpu.VMEM((NB, IB), jnp.int32),
            pltpu.VMEM((CW, D), jnp.float32),
            pltpu.VMEM((W,), jnp.float32),
            pltpu.VMEM((W,), jnp.float32),
            pltpu.SemaphoreType.DMA,
            pltpu.SemaphoreType.DMA,
            pltpu.SemaphoreType.DMA,
            pltpu.SemaphoreType.DMA,
        ],
    )
    return f(words, pc2d, nc2d, emb_table, out_table)


def _loss_body(ps_ref, ns_ref, o_ref):
    ps = ps_ref[...]
    ns = ns_ref[...]
    pos_loss = jax.nn.log_sigmoid(ps)
    neg_loss = jax.nn.log_sigmoid(-ns)
    o_ref[0, 0] = -jnp.sum(pos_loss + neg_loss) / jnp.float32(N)


@jax.jit
def _tc_loss(ps, ns):
    out = pl.pallas_call(
        _loss_body,
        out_shape=jax.ShapeDtypeStruct((1, 1), jnp.float32),
        out_specs=pl.BlockSpec(memory_space=pltpu.SMEM),
    )(ps.reshape(128, 128), ns.reshape(128, 128))
    return out[0, 0]


def kernel(words, pos_contexts, neg_contexts, emb_table, out_table):
    words = words.astype(jnp.int32)
    pc_flat = pos_contexts.astype(jnp.int32).reshape(N * C)
    nc_flat = neg_contexts.astype(jnp.int32).reshape(N * C)
    ps, ns = _sc_sims(words, pc_flat, nc_flat, emb_table, out_table)
    return _tc_loss(ps, ns)


# speedup vs baseline: 1.0635x; 1.0635x over previous
import jax
import jax.numpy as jnp
from jax import lax
from jax.experimental import pallas as pl
from jax.experimental.pallas import tpu as pltpu
from jax.experimental.pallas import tpu_sc as plsc

V, D, N, C = 1000000, 64, 16384, 20

NC, NS = 2, 16
NW = NC * NS
WPW = N // NW
W = 16
CW = W * C
NCH = WPW // W
NCHT = 2 * NCH
NREG = D // 16
ROW = 128


def _sc_body(wr_hbm, wh_hbm, cr_hbm, ch_hbm, emb2_hbm, out2_hbm,
             sims_hbm,
             wrv, whv, rbuf, hbuf, cbuf, tbuf, simbuf,
             sem_i0, sem_i1, sem_d0, sem_d1):
    wid = lax.axis_index("s") * NC + lax.axis_index("c")
    base = wid * WPW
    sem_i = (sem_i0, sem_i1)
    sem_d = (sem_d0, sem_d1)

    def ctx_off(g):
        p = jnp.asarray(g >= NCH, jnp.int32)
        return p * (N * C) + (base + (g - p * NCH) * W) * C

    def issue_idx(g, b):
        co = ctx_off(g)
        pltpu.async_copy(cr_hbm.at[pl.ds(co, CW)], rbuf.at[b], sem_i[b])
        pltpu.async_copy(ch_hbm.at[pl.ds(co, CW)], hbuf.at[b], sem_i[b])

    def wait_idx(b):
        pltpu.make_async_copy(cr_hbm.at[pl.ds(0, CW)], rbuf.at[b], sem_i[b]).wait()
        pltpu.make_async_copy(ch_hbm.at[pl.ds(0, CW)], hbuf.at[b], sem_i[b]).wait()

    def issue_gathers(g, b):
        wl = lax.rem(g, NCH) * W
        pltpu.async_copy(out2_hbm.at[rbuf.at[b, pl.ds(0, 128)]],
                         cbuf.at[b, pl.ds(0, 128)], sem_d[b])
        pltpu.async_copy(out2_hbm.at[rbuf.at[b, pl.ds(128, 128)]],
                         cbuf.at[b, pl.ds(128, 128)], sem_d[b])
        pltpu.async_copy(out2_hbm.at[rbuf.at[b, pl.ds(256, 64)]],
                         cbuf.at[b, pl.ds(256, 64)], sem_d[b])
        pltpu.async_copy(emb2_hbm.at[wrv.at[pl.ds(wl, W)]], tbuf.at[b], sem_d[b])

    def wait_data(b):
        pltpu.make_async_copy(out2_hbm.at[pl.ds(0, CW)], cbuf.at[b], sem_d[b]).wait()
        pltpu.make_async_copy(emb2_hbm.at[pl.ds(0, W)], tbuf.at[b], sem_d[b]).wait()

    def compute(g, b):
        wl = lax.rem(g, NCH) * W

        def word_body(w, sv):
            r0 = w * C
            hv1 = hbuf[b, pl.ds(r0, 16)]
            hv2 = hbuf[b, pl.ds(r0 + 4, 16)]
            acc = [None] * NREG
            for c in range(C):
                hc = hv1[c] if c < 16 else hv2[c - 4]
                hc = pl.multiple_of(hc, 64)
                for j in range(NREG):
                    x = cbuf[b, r0 + c, pl.ds(hc + j * 16, 16)]
                    acc[j] = x if acc[j] is None else acc[j] + x
            thv = whv[pl.ds(wl + w, 16)]
            th = pl.multiple_of(thv[0], 64)
            prod = [acc[j] * tbuf[b, w, pl.ds(th + j * 16, 16)] for j in range(NREG)]
            q = (prod[0] + prod[1]) + (prod[2] + prod[3])
            for k in (1, 2, 4, 8):
                q = q + jnp.take_along_axis(q, jnp.bitwise_xor(lanes, k), axis=0)
            return jnp.where(lanes == w, q, sv)

        lanes = lax.iota(jnp.int32, 16)
        sv = lax.fori_loop(0, W, word_body, jnp.zeros((16,), jnp.float32),
                           unroll=False)
        simbuf[pl.ds(g * W, W)] = sv

    pltpu.sync_copy(wr_hbm.at[pl.ds(base, WPW)], wrv)
    pltpu.sync_copy(wh_hbm.at[pl.ds(base, WPW)], whv.at[pl.ds(0, WPW)])
    issue_idx(0, 0)
    issue_idx(1, 1)
    wait_idx(0)
    issue_gathers(0, 0)

    def chunk_pair(gg, _):
        for sub in range(2):
            g = gg * 2 + sub
            b = sub
            nb = 1 - sub

            @pl.when(g < NCHT - 1)
            def _():
                wait_idx(nb)
                issue_gathers(g + 1, nb)

            wait_data(b)
            compute(g, b)

            @pl.when(g < NCHT - 2)
            def _():
                issue_idx(g + 2, b)

        return 0

    lax.fori_loop(0, NCHT // 2, chunk_pair, 0, unroll=False)

    pltpu.sync_copy(simbuf.at[pl.ds(0, WPW)], sims_hbm.at[pl.ds(base, WPW)])
    pltpu.sync_copy(simbuf.at[pl.ds(WPW, WPW)],
                    sims_hbm.at[pl.ds(N + base, WPW)])


@jax.jit
def _sc_sims(words, pos_contexts, neg_contexts, emb_table, out_table):
    pc = pos_contexts.astype(jnp.int32).reshape(N * C)
    nc = neg_contexts.astype(jnp.int32).reshape(N * C)
    wi = words.astype(jnp.int32)
    cr = jnp.concatenate([pc >> 1, nc >> 1])
    ch = jnp.concatenate([(pc & 1) << 6, (nc & 1) << 6])
    wr = wi >> 1
    wh = (wi & 1) << 6
    emb2 = emb_table.reshape(V // 2, ROW)
    out2 = out_table.reshape(V // 2, ROW)

    mesh = plsc.VectorSubcoreMesh(core_axis_name="c", subcore_axis_name="s",
                                  num_cores=NC, num_subcores=NS)
    f = pl.kernel(
        _sc_body,
        out_type=jax.ShapeDtypeStruct((2 * N,), jnp.float32),
        mesh=mesh,
        compiler_params=pltpu.CompilerParams(use_tc_tiling_on_sc=False),
        scratch_types=[
            pltpu.VMEM((WPW,), jnp.int32),
            pltpu.VMEM((WPW + 16,), jnp.int32),
            pltpu.VMEM((2, CW), jnp.int32),
            pltpu.VMEM((2, CW), jnp.int32),
            pltpu.VMEM((2, CW, ROW), jnp.float32),
            pltpu.VMEM((2, W, ROW), jnp.float32),
            pltpu.VMEM((2 * WPW,), jnp.float32),
            pltpu.SemaphoreType.DMA,
            pltpu.SemaphoreType.DMA,
            pltpu.SemaphoreType.DMA,
            pltpu.SemaphoreType.DMA,
        ],
    )
    return f(wr, wh, cr, ch, emb2, out2)


def _loss_body(s_ref, o_ref):
    s = s_ref[...]
    ps = s[0:128, :]
    ns = s[128:256, :]
    pos_loss = jax.nn.log_sigmoid(ps)
    neg_loss = jax.nn.log_sigmoid(-ns)
    o_ref[0, 0] = -jnp.sum(pos_loss + neg_loss) / jnp.float32(N)


@jax.jit
def _tc_loss(sims):
    out = pl.pallas_call(
        _loss_body,
        out_shape=jax.ShapeDtypeStruct((1, 1), jnp.float32),
        out_specs=pl.BlockSpec(memory_space=pltpu.SMEM),
    )(sims.reshape(256, 128))
    return out[0, 0]


def kernel(words, pos_contexts, neg_contexts, emb_table, out_table):
    sims = _sc_sims(words, pos_contexts, neg_contexts, emb_table, out_table)
    return _tc_loss(sims)


# speedup vs baseline: 1.3941x; 1.3108x over previous
import jax
import jax.numpy as jnp
from jax import lax
from jax.experimental import pallas as pl
from jax.experimental.pallas import tpu as pltpu
from jax.experimental.pallas import tpu_sc as plsc

V, D, N, C = 1000000, 64, 16384, 20

NC, NS = 2, 16
NW = NC * NS
WPW = N // NW
W = 16
CW = W * C
NCH = WPW // W
NREG = D // 16


def _sc_body(temb_hbm, pcf_hbm, ncf_hbm, out_hbm,
             sims_hbm,
             rbuf, cbuf, tbuf, simbuf,
             sem_i0, sem_i1, sem_d0, sem_d1):
    wid = lax.axis_index("s") * NC + lax.axis_index("c")
    base = wid * WPW
    sem_i = (sem_i0, sem_i1)
    sem_d = (sem_d0, sem_d1)

    def run_phase(cr_hbm, ph):
        def issue_idx(g, b):
            co = (base + g * W) * C
            pltpu.async_copy(cr_hbm.at[pl.ds(co, CW)], rbuf.at[b], sem_i[b])

        def wait_idx(b):
            pltpu.make_async_copy(cr_hbm.at[pl.ds(0, CW)], rbuf.at[b],
                                  sem_i[b]).wait()

        def issue_gathers(g, b):
            wb = base + g * W
            pltpu.async_copy(out_hbm.at[rbuf.at[b, pl.ds(0, 128)]],
                             cbuf.at[b, pl.ds(0, 128)], sem_d[b])
            pltpu.async_copy(out_hbm.at[rbuf.at[b, pl.ds(128, 128)]],
                             cbuf.at[b, pl.ds(128, 128)], sem_d[b])
            pltpu.async_copy(out_hbm.at[rbuf.at[b, pl.ds(256, 64)]],
                             cbuf.at[b, pl.ds(256, 64)], sem_d[b])
            pltpu.async_copy(temb_hbm.at[pl.ds(wb, W), :], tbuf.at[b],
                             sem_d[b])

        def wait_data(b):
            pltpu.make_async_copy(out_hbm.at[pl.ds(0, CW)], cbuf.at[b],
                                  sem_d[b]).wait()
            pltpu.make_async_copy(temb_hbm.at[pl.ds(0, W), :], tbuf.at[b],
                                  sem_d[b]).wait()

        def compute(g, b):
            lanes = lax.iota(jnp.int32, 16)

            def word_body(w, sv):
                r0 = w * C
                acc = [None] * NREG
                for c in range(C):
                    for j in range(NREG):
                        x = cbuf[b, r0 + c, pl.ds(j * 16, 16)]
                        acc[j] = x if acc[j] is None else acc[j] + x
                prod = [acc[j] * tbuf[b, w, pl.ds(j * 16, 16)]
                        for j in range(NREG)]
                q = (prod[0] + prod[1]) + (prod[2] + prod[3])
                for k in (1, 2, 4, 8):
                    q = q + jnp.take_along_axis(
                        q, jnp.bitwise_xor(lanes, k), axis=0)
                return jnp.where(lanes == w, q, sv)

            sv = lax.fori_loop(0, W, word_body, jnp.zeros((16,), jnp.float32),
                               unroll=False)
            simbuf[pl.ds(ph * WPW + g * W, W)] = sv

        issue_idx(0, 0)
        issue_idx(1, 1)
        wait_idx(0)
        issue_gathers(0, 0)

        def chunk_pair(gg, _):
            for sub in range(2):
                g = gg * 2 + sub
                b = sub
                nb = 1 - sub

                @pl.when(g < NCH - 1)
                def _():
                    wait_idx(nb)
                    issue_gathers(g + 1, nb)

                wait_data(b)
                compute(g, b)

                @pl.when(g < NCH - 2)
                def _():
                    issue_idx(g + 2, b)

            return 0

        lax.fori_loop(0, NCH // 2, chunk_pair, 0, unroll=False)

    run_phase(pcf_hbm, 0)
    run_phase(ncf_hbm, 1)

    pltpu.sync_copy(simbuf.at[pl.ds(0, WPW)], sims_hbm.at[pl.ds(base, WPW)])
    pltpu.sync_copy(simbuf.at[pl.ds(WPW, WPW)],
                    sims_hbm.at[pl.ds(N + base, WPW)])


@jax.jit
def _sc_sims(words, pos_contexts, neg_contexts, emb_table, out_table):
    wi = words.astype(jnp.int32)
    pcf = pos_contexts.astype(jnp.int32).reshape(N * C)
    ncf = neg_contexts.astype(jnp.int32).reshape(N * C)
    temb = jnp.take(emb_table, wi, axis=0)

    mesh = plsc.VectorSubcoreMesh(core_axis_name="c", subcore_axis_name="s",
                                  num_cores=NC, num_subcores=NS)
    f = pl.kernel(
        _sc_body,
        out_type=jax.ShapeDtypeStruct((2 * N,), jnp.float32),
        mesh=mesh,
        compiler_params=pltpu.CompilerParams(use_tc_tiling_on_sc=False),
        scratch_types=[
            pltpu.VMEM((2, CW), jnp.int32),
            pltpu.VMEM((2, CW, D), jnp.float32),
            pltpu.VMEM((2, W, D), jnp.float32),
            pltpu.VMEM((2 * WPW,), jnp.float32),
            pltpu.SemaphoreType.DMA,
            pltpu.SemaphoreType.DMA,
            pltpu.SemaphoreType.DMA,
            pltpu.SemaphoreType.DMA,
        ],
    )
    return f(temb, pcf, ncf, out_table)


def _loss_body(s_ref, o_ref):
    s = s_ref[...]
    ps = s[0:128, :]
    ns = s[128:256, :]
    pos_loss = jax.nn.log_sigmoid(ps)
    neg_loss = jax.nn.log_sigmoid(-ns)
    o_ref[0, 0] = -jnp.sum(pos_loss + neg_loss) / jnp.float32(N)


@jax.jit
def _tc_loss(sims):
    out = pl.pallas_call(
        _loss_body,
        out_shape=jax.ShapeDtypeStruct((1, 1), jnp.float32),
        out_specs=pl.BlockSpec(memory_space=pltpu.SMEM),
    )(sims.reshape(256, 128))
    return out[0, 0]


def kernel(words, pos_contexts, neg_contexts, emb_table, out_table):
    sims = _sc_sims(words, pos_contexts, neg_contexts, emb_table, out_table)
    return _tc_loss(sims)


# speedup vs baseline: 1.9592x; 1.4054x over previous
import jax
import jax.numpy as jnp
from jax import lax
from jax.experimental import pallas as pl
from jax.experimental.pallas import tpu as pltpu
from jax.experimental.pallas import tpu_sc as plsc

V, D, N, C = 1000000, 64, 16384, 20

NC, NS = 2, 16
NW = NC * NS
WPW = N // NW
W = 16
CW = W * C
NCH = WPW // W
NREG = D // 16
ROW = 2 * D
TB = 16384
HB = TB // 2
NTB = -(-V // TB)
WIDE_R = NTB * HB


def _tr_body(x_ref, o_ref):
    o_ref[:, 0:D] = x_ref[:, 0:HB].T
    o_ref[:, D:ROW] = x_ref[:, HB:TB].T


@jax.jit
def _widen(out_table):
    t = out_table.T
    return pl.pallas_call(
        _tr_body,
        grid=(NTB,),
        in_specs=[pl.BlockSpec((D, TB), lambda i: (0, i))],
        out_specs=pl.BlockSpec((HB, ROW), lambda i: (i, 0)),
        out_shape=jax.ShapeDtypeStruct((WIDE_R, ROW), jnp.float32),
    )(t)


def _sc_body(temb_hbm, pcf_hbm, ncf_hbm, phf_hbm, nhf_hbm, wide_hbm,
             sims_hbm,
             rbuf, hbuf, cbuf, tbuf, simbuf,
             sem_i0, sem_i1, sem_d0, sem_d1):
    wid = lax.axis_index("s") * NC + lax.axis_index("c")
    base = wid * WPW
    sem_i = (sem_i0, sem_i1)
    sem_d = (sem_d0, sem_d1)

    def run_phase(cr_hbm, ch_hbm, ph):
        def issue_idx(g, b):
            co = (base + g * W) * C
            pltpu.async_copy(cr_hbm.at[pl.ds(co, CW)], rbuf.at[b], sem_i[b])
            pltpu.async_copy(ch_hbm.at[pl.ds(co, CW)], hbuf.at[b], sem_i[b])

        def wait_idx(b):
            pltpu.make_async_copy(cr_hbm.at[pl.ds(0, CW)], rbuf.at[b],
                                  sem_i[b]).wait()
            pltpu.make_async_copy(ch_hbm.at[pl.ds(0, CW)], hbuf.at[b],
                                  sem_i[b]).wait()

        def issue_gathers(g, b):
            wb = base + g * W
            pltpu.async_copy(wide_hbm.at[rbuf.at[b, pl.ds(0, 128)]],
                             cbuf.at[b, pl.ds(0, 128)], sem_d[b])
            pltpu.async_copy(wide_hbm.at[rbuf.at[b, pl.ds(128, 128)]],
                             cbuf.at[b, pl.ds(128, 128)], sem_d[b])
            pltpu.async_copy(wide_hbm.at[rbuf.at[b, pl.ds(256, 64)]],
                             cbuf.at[b, pl.ds(256, 64)], sem_d[b])
            pltpu.async_copy(temb_hbm.at[pl.ds(wb, W), :], tbuf.at[b],
                             sem_d[b])

        def wait_data(b):
            pltpu.make_async_copy(wide_hbm.at[pl.ds(0, CW)], cbuf.at[b],
                                  sem_d[b]).wait()
            pltpu.make_async_copy(temb_hbm.at[pl.ds(0, W), :], tbuf.at[b],
                                  sem_d[b]).wait()

        def compute(g, b):
            lanes = lax.iota(jnp.int32, 16)

            def word_body(w, sv):
                r0 = w * C
                hv1 = hbuf[b, pl.ds(r0, 16)]
                hv2 = hbuf[b, pl.ds(r0 + 4, 16)]
                acc = [None] * NREG
                for c in range(C):
                    hc = hv1[c] if c < 16 else hv2[c - 4]
                    hc = pl.multiple_of(hc, 64)
                    for j in range(NREG):
                        x = cbuf[b, r0 + c, pl.ds(hc + j * 16, 16)]
                        acc[j] = x if acc[j] is None else acc[j] + x
                prod = [acc[j] * tbuf[b, w, pl.ds(j * 16, 16)]
                        for j in range(NREG)]
                q = (prod[0] + prod[1]) + (prod[2] + prod[3])
                for k in (1, 2, 4, 8):
                    q = q + jnp.take_along_axis(
                        q, jnp.bitwise_xor(lanes, k), axis=0)
                return jnp.where(lanes == w, q, sv)

            sv = lax.fori_loop(0, W, word_body, jnp.zeros((16,), jnp.float32),
                               unroll=False)
            simbuf[pl.ds(ph * WPW + g * W, W)] = sv

        issue_idx(0, 0)
        issue_idx(1, 1)
        wait_idx(0)
        issue_gathers(0, 0)

        def chunk_pair(gg, _):
            for sub in range(2):
                g = gg * 2 + sub
                b = sub
                nb = 1 - sub

                @pl.when(g < NCH - 1)
                def _():
                    wait_idx(nb)
                    issue_gathers(g + 1, nb)

                wait_data(b)
                compute(g, b)

                @pl.when(g < NCH - 2)
                def _():
                    issue_idx(g + 2, b)

            return 0

        lax.fori_loop(0, NCH // 2, chunk_pair, 0, unroll=False)

    run_phase(pcf_hbm, phf_hbm, 0)
    run_phase(ncf_hbm, nhf_hbm, 1)

    pltpu.sync_copy(simbuf.at[pl.ds(0, WPW)], sims_hbm.at[pl.ds(base, WPW)])
    pltpu.sync_copy(simbuf.at[pl.ds(WPW, WPW)],
                    sims_hbm.at[pl.ds(N + base, WPW)])


@jax.jit
def _sc_sims(words, pos_contexts, neg_contexts, emb_table, out_table):
    wi = words.astype(jnp.int32)
    pc = pos_contexts.astype(jnp.int32).reshape(N * C)
    ngc = neg_contexts.astype(jnp.int32).reshape(N * C)
    pcf = ((pc >> 14) << 13) + (pc & (HB - 1))
    ncf = ((ngc >> 14) << 13) + (ngc & (HB - 1))
    phf = ((pc >> 13) & 1) * D
    nhf = ((ngc >> 13) & 1) * D
    temb = jnp.take(emb_table, wi, axis=0)
    wide = _widen(out_table)

    mesh = plsc.VectorSubcoreMesh(core_axis_name="c", subcore_axis_name="s",
                                  num_cores=NC, num_subcores=NS)
    f = pl.kernel(
        _sc_body,
        out_type=jax.ShapeDtypeStruct((2 * N,), jnp.float32),
        mesh=mesh,
        compiler_params=pltpu.CompilerParams(use_tc_tiling_on_sc=False),
        scratch_types=[
            pltpu.VMEM((2, CW), jnp.int32),
            pltpu.VMEM((2, CW), jnp.int32),
            pltpu.VMEM((2, CW, ROW), jnp.float32),
            pltpu.VMEM((2, W, D), jnp.float32),
            pltpu.VMEM((2 * WPW,), jnp.float32),
            pltpu.SemaphoreType.DMA,
            pltpu.SemaphoreType.DMA,
            pltpu.SemaphoreType.DMA,
            pltpu.SemaphoreType.DMA,
        ],
    )
    return f(temb, pcf, ncf, phf, nhf, wide)


def _loss_body(s_ref, o_ref):
    s = s_ref[...]
    ps = s[0:128, :]
    ns = s[128:256, :]
    pos_loss = jax.nn.log_sigmoid(ps)
    neg_loss = jax.nn.log_sigmoid(-ns)
    o_ref[0, 0] = -jnp.sum(pos_loss + neg_loss) / jnp.float32(N)


@jax.jit
def _tc_loss(sims):
    out = pl.pallas_call(
        _loss_body,
        out_shape=jax.ShapeDtypeStruct((1, 1), jnp.float32),
        out_specs=pl.BlockSpec(memory_space=pltpu.SMEM),
    )(sims.reshape(256, 128))
    return out[0, 0]


def kernel(words, pos_contexts, neg_contexts, emb_table, out_table):
    sims = _sc_sims(words, pos_contexts, neg_contexts, emb_table, out_table)
    return _tc_loss(sims)
